# baseline (speedup 1.0000x reference)
import functools

import jax
import jax.numpy as jnp
from jax import lax
from jax.experimental import pallas as pl
from jax.experimental.pallas import tpu as pltpu
from jax.experimental.pallas import tpu_sc as plsc

EMBED_DIM = 32
NUM_CORES = 2
NUM_SUBCORES = 16
NW = NUM_CORES * NUM_SUBCORES
B_B1 = 16
L = 16
PAD_B2 = 32


@functools.lru_cache(maxsize=None)
def _make_gather(n_b1: int, n_b2: int):
    rows_per_w = n_b1 // NW
    n_blocks = rows_per_w // B_B1

    mesh = plsc.VectorSubcoreMesh(
        core_axis_name="c", subcore_axis_name="s",
        num_cores=NUM_CORES, num_subcores=NUM_SUBCORES)

    @functools.partial(
        pl.kernel,
        mesh=mesh,
        out_type=jax.ShapeDtypeStruct((n_b2, EMBED_DIM, n_b1), jnp.float32),
        scratch_types=[
            pltpu.VMEM((rows_per_w, PAD_B2), jnp.int32),
            pltpu.VMEM((2, B_B1, PAD_B2, EMBED_DIM), jnp.float32),
            pltpu.VMEM((2, n_b2, EMBED_DIM, B_B1), jnp.float32),
            pltpu.SemaphoreType.DMA,
            pltpu.SemaphoreType.DMA,
        ],
        compiler_params=pltpu.CompilerParams(
            use_tc_tiling_on_sc=False, needs_layout_passes=False),
    )
    def gather_kernel(idx_hbm, table_hbm, out_hbm, idx_v, rows_v, tbuf,
                      gsem, osem):
        wid = lax.axis_index("s") * NUM_CORES + lax.axis_index("c")
        pltpu.sync_copy(idx_hbm.at[pl.ds(wid * rows_per_w, rows_per_w)],
                        idx_v)
        iota = lax.iota(jnp.int32, L)

        def fire(b, slot):
            @pl.loop(0, B_B1)
            def _fire_row(r):
                pltpu.async_copy(
                    table_hbm.at[idx_v.at[b * B_B1 + r]],
                    rows_v.at[slot, r], gsem)

        def drain(slot):
            @pl.loop(0, B_B1)
            def _drain_row(r):
                pltpu.make_async_copy(
                    table_hbm.at[idx_v.at[0]],
                    rows_v.at[slot, 0], gsem).wait()

        def out_dst(b):
            b1o = wid * rows_per_w + b * B_B1
            return out_hbm.at[:, :, pl.ds(b1o, B_B1)]

        def owait():
            pltpu.make_async_copy(tbuf.at[0], out_dst(0), osem).wait()

        def transpose(slot):
            for xr in range(B_B1):
                xrv = jnp.full((L,), xr, jnp.int32)
                for b2 in range(n_b2):
                    b2v = jnp.full((L,), b2, jnp.int32)
                    for d0 in range(0, EMBED_DIM, L):
                        vals = rows_v[slot, xr, b2, pl.ds(d0, L)]
                        plsc.store_scatter(
                            tbuf.at[slot], [b2v, iota + d0, xrv], vals)

        fire(0, 0)

        @pl.loop(0, n_blocks)
        def _block(b):
            slot = lax.rem(b, 2)
            drain(slot)

            @pl.when(b + 1 < n_blocks)
            def _fire_next():
                fire(b + 1, lax.rem(b + 1, 2))

            @pl.when(b >= 2)
            def _wait_out():
                owait()

            transpose(slot)
            pltpu.async_copy(tbuf.at[slot], out_dst(b), osem)

        owait()
        owait()

    return gather_kernel


def kernel(x, table):
    n_b1, n_b2 = x.shape
    xp = jnp.pad(x.astype(jnp.int32), ((0, 0), (0, PAD_B2 - n_b2)))
    out = _make_gather(n_b1, n_b2)(xp, table)
    return jnp.transpose(out, (2, 0, 1))


# speedup vs baseline: 1.9676x; 1.9676x over previous
import functools

import jax
import jax.numpy as jnp
from jax import lax
from jax.experimental import pallas as pl
from jax.experimental.pallas import tpu as pltpu
from jax.experimental.pallas import tpu_sc as plsc

EMBED_DIM = 32
NUM_CORES = 2
NUM_SUBCORES = 16
NW = NUM_CORES * NUM_SUBCORES
IDX_LANE = 128
L = 16
PAD_B2 = 32


@functools.lru_cache(maxsize=None)
def _make_gather(n_b1: int, n_b2: int):
    cols_per_w = n_b1 // NW
    VPB = cols_per_w // IDX_LANE

    mesh = plsc.VectorSubcoreMesh(
        core_axis_name="c", subcore_axis_name="s",
        num_cores=NUM_CORES, num_subcores=NUM_SUBCORES)

    @functools.partial(
        pl.kernel,
        mesh=mesh,
        out_type=jax.ShapeDtypeStruct((n_b2, EMBED_DIM, n_b1), jnp.float32),
        scratch_types=[
            pltpu.VMEM((PAD_B2, cols_per_w), jnp.int32),
            pltpu.VMEM((2, cols_per_w, EMBED_DIM), jnp.float32),
            pltpu.VMEM((2, EMBED_DIM, cols_per_w), jnp.float32),
            pltpu.SemaphoreType.DMA,
            pltpu.SemaphoreType.DMA,
        ],
        compiler_params=pltpu.CompilerParams(
            use_tc_tiling_on_sc=False, needs_layout_passes=False),
    )
    def gather_kernel(idx_hbm, table_hbm, out_hbm, idx_v, rows_v, tbuf,
                      gsem, osem):
        wid = lax.axis_index("s") * NUM_CORES + lax.axis_index("c")
        c0 = wid * cols_per_w
        pltpu.sync_copy(idx_hbm.at[:, pl.ds(c0, cols_per_w)], idx_v)
        iota = lax.iota(jnp.int32, L)

        def fire(b2, slot):
            for k in range(VPB):
                pltpu.async_copy(
                    table_hbm.at[idx_v.at[b2, pl.ds(k * IDX_LANE, IDX_LANE)]],
                    rows_v.at[slot].at[pl.ds(k * IDX_LANE, IDX_LANE)],
                    gsem)

        def drain(slot):
            for k in range(VPB):
                pltpu.make_async_copy(
                    table_hbm.at[idx_v.at[0, pl.ds(0, IDX_LANE)]],
                    rows_v.at[slot].at[pl.ds(0, IDX_LANE)],
                    gsem).wait()

        def out_dst(b2):
            return out_hbm.at[b2, :, pl.ds(c0, cols_per_w)]

        def owait():
            pltpu.make_async_copy(tbuf.at[0], out_dst(0), osem).wait()

        def transpose(slot):
            @pl.loop(0, EMBED_DIM)
            def _d(d):
                dv = jnp.full((L,), d, jnp.int32)
                for g in range(cols_per_w // L):
                    cv = iota + g * L
                    tbuf.at[slot, d][pl.ds(g * L, L)] = plsc.load_gather(
                        rows_v.at[slot], [cv, dv])

        fire(0, 0)

        @pl.loop(0, n_b2)
        def _b2(b2):
            slot = lax.rem(b2, 2)
            drain(slot)

            @pl.when(b2 + 1 < n_b2)
            def _fire_next():
                fire(b2 + 1, lax.rem(b2 + 1, 2))

            @pl.when(b2 >= 2)
            def _wait_out():
                owait()

            transpose(slot)
            pltpu.async_copy(tbuf.at[slot], out_dst(b2), osem)

        owait()
        owait()

    return gather_kernel


def kernel(x, table):
    n_b1, n_b2 = x.shape
    xt = jnp.pad(x.astype(jnp.int32).T, ((0, PAD_B2 - n_b2), (0, 0)))
    out = _make_gather(n_b1, n_b2)(xt, table)
    return jnp.transpose(out, (2, 0, 1))


# speedup vs baseline: 2.0768x; 1.0555x over previous
import functools

import jax
import jax.numpy as jnp
from jax import lax
from jax.experimental import pallas as pl
from jax.experimental.pallas import tpu as pltpu
from jax.experimental.pallas import tpu_sc as plsc

EMBED_DIM = 32
NUM_CORES = 2
NUM_SUBCORES = 16
NW = NUM_CORES * NUM_SUBCORES
IDX_LANE = 128
L = 16
PAD_B2 = 32


@functools.lru_cache(maxsize=None)
def _make_gather(n_b1: int, n_b2: int):
    cols_per_w = n_b1 // NW
    VPB = cols_per_w // IDX_LANE

    mesh = plsc.VectorSubcoreMesh(
        core_axis_name="c", subcore_axis_name="s",
        num_cores=NUM_CORES, num_subcores=NUM_SUBCORES)

    @functools.partial(
        pl.kernel,
        mesh=mesh,
        out_type=jax.ShapeDtypeStruct((n_b2, EMBED_DIM, n_b1), jnp.float32),
        scratch_types=[
            pltpu.VMEM((n_b2, cols_per_w), jnp.int32),
            pltpu.VMEM((2, cols_per_w, EMBED_DIM), jnp.float32),
            pltpu.VMEM((2, EMBED_DIM, cols_per_w), jnp.float32),
            pltpu.SemaphoreType.DMA,
            pltpu.SemaphoreType.DMA,
        ],
        compiler_params=pltpu.CompilerParams(
            use_tc_tiling_on_sc=False, needs_layout_passes=False),
    )
    def gather_kernel(idx_hbm, table_hbm, out_hbm, idx_v, rows_v, tbuf,
                      gsem, osem):
        wid = lax.axis_index("s") * NUM_CORES + lax.axis_index("c")
        c0 = wid * cols_per_w
        for b2s in range(n_b2):
            pltpu.sync_copy(idx_hbm.at[pl.ds(b2s * n_b1 + c0, cols_per_w)],
                            idx_v.at[b2s])
        iota = lax.iota(jnp.int32, L)

        def fire(b2, slot):
            for k in range(VPB):
                pltpu.async_copy(
                    table_hbm.at[idx_v.at[b2, pl.ds(k * IDX_LANE, IDX_LANE)]],
                    rows_v.at[slot].at[pl.ds(k * IDX_LANE, IDX_LANE)],
                    gsem)

        def drain(slot):
            for k in range(VPB):
                pltpu.make_async_copy(
                    table_hbm.at[idx_v.at[0, pl.ds(0, IDX_LANE)]],
                    rows_v.at[slot].at[pl.ds(0, IDX_LANE)],
                    gsem).wait()

        def out_dst(b2):
            return out_hbm.at[b2, :, pl.ds(c0, cols_per_w)]

        def owait():
            pltpu.make_async_copy(tbuf.at[0], out_dst(0), osem).wait()

        def transpose(slot):
            @pl.loop(0, EMBED_DIM)
            def _d(d):
                dv = jnp.full((L,), d, jnp.int32)
                for g0 in range(0, cols_per_w // L, 8):
                    cvs = [iota + (g0 + j) * L for j in range(8)]
                    vals = [plsc.load_gather(rows_v.at[slot], [cvs[j], dv])
                            for j in range(8)]
                    for j in range(8):
                        tbuf.at[slot, d][pl.ds((g0 + j) * L, L)] = vals[j]

        fire(0, 0)

        @pl.loop(0, n_b2)
        def _b2(b2):
            slot = lax.rem(b2, 2)
            drain(slot)

            @pl.when(b2 + 1 < n_b2)
            def _fire_next():
                fire(b2 + 1, lax.rem(b2 + 1, 2))

            @pl.when(b2 >= 2)
            def _wait_out():
                owait()

            transpose(slot)
            pltpu.async_copy(tbuf.at[slot], out_dst(b2), osem)

        owait()
        owait()

    return gather_kernel


def kernel(x, table):
    n_b1, n_b2 = x.shape
    xt = jnp.pad(x.astype(jnp.int32).T, ((0, PAD_B2 - n_b2), (0, 0)))
    out = _make_gather(n_b1, n_b2)(xt.reshape(-1), table)
    return jnp.transpose(out, (2, 0, 1))


# speedup vs baseline: 2.1053x; 1.0137x over previous
import functools

import jax
import jax.numpy as jnp
from jax import lax
from jax.experimental import pallas as pl
from jax.experimental.pallas import tpu as pltpu
from jax.experimental.pallas import tpu_sc as plsc

EMBED_DIM = 32
NUM_CORES = 2
NUM_SUBCORES = 16
NW = NUM_CORES * NUM_SUBCORES
IDX_LANE = 128
L = 16
PAD_B2 = 32


@functools.lru_cache(maxsize=None)
def _make_gather(n_b1: int, n_b2: int):
    cols_per_w = n_b1 // NW
    VPB = cols_per_w // IDX_LANE

    mesh = plsc.VectorSubcoreMesh(
        core_axis_name="c", subcore_axis_name="s",
        num_cores=NUM_CORES, num_subcores=NUM_SUBCORES)

    @functools.partial(
        pl.kernel,
        mesh=mesh,
        out_type=jax.ShapeDtypeStruct((n_b2, EMBED_DIM, n_b1), jnp.float32),
        scratch_types=[
            pltpu.VMEM((PAD_B2 // 8, cols_per_w // IDX_LANE, 8, IDX_LANE),
                       jnp.int32),
            pltpu.VMEM((2, cols_per_w, EMBED_DIM), jnp.float32),
            pltpu.VMEM((2, EMBED_DIM, cols_per_w), jnp.float32),
            pltpu.SemaphoreType.DMA,
            pltpu.SemaphoreType.DMA,
        ],
        compiler_params=pltpu.CompilerParams(
            use_tc_tiling_on_sc=False, needs_layout_passes=False),
    )
    def gather_kernel(idx_hbm, table_hbm, out_hbm, idx_v, rows_v, tbuf,
                      gsem, osem):
        wid = lax.axis_index("s") * NUM_CORES + lax.axis_index("c")
        c0 = wid * cols_per_w
        tc0 = wid * (cols_per_w // IDX_LANE)
        pltpu.sync_copy(
            idx_hbm.at[:, pl.ds(tc0, cols_per_w // IDX_LANE)], idx_v)
        iota = lax.iota(jnp.int32, L)

        def fire(b2, slot):
            tr = b2 // 8
            r = b2 % 8
            for k in range(VPB):
                pltpu.async_copy(
                    table_hbm.at[idx_v.at[tr, k, r]],
                    rows_v.at[slot].at[pl.ds(k * IDX_LANE, IDX_LANE)],
                    gsem)

        def drain(slot):
            for k in range(VPB):
                pltpu.make_async_copy(
                    table_hbm.at[idx_v.at[0, 0, 0]],
                    rows_v.at[slot].at[pl.ds(0, IDX_LANE)],
                    gsem).wait()

        def out_dst(b2):
            return out_hbm.at[b2, :, pl.ds(c0, cols_per_w)]

        def owait():
            pltpu.make_async_copy(tbuf.at[0], out_dst(0), osem).wait()

        def transpose(slot):
            @pl.loop(0, EMBED_DIM)
            def _d(d):
                dv = jnp.full((L,), d, jnp.int32)
                for g0 in range(0, cols_per_w // L, 8):
                    cvs = [iota + (g0 + j) * L for j in range(8)]
                    vals = [plsc.load_gather(rows_v.at[slot], [cvs[j], dv])
                            for j in range(8)]
                    for j in range(8):
                        tbuf.at[slot, d][pl.ds((g0 + j) * L, L)] = vals[j]

        fire(0, 0)

        @pl.loop(0, n_b2)
        def _b2(b2):
            slot = lax.rem(b2, 2)
            drain(slot)

            @pl.when(b2 + 1 < n_b2)
            def _fire_next():
                fire(b2 + 1, lax.rem(b2 + 1, 2))

            @pl.when(b2 >= 2)
            def _wait_out():
                owait()

            transpose(slot)
            pltpu.async_copy(tbuf.at[slot], out_dst(b2), osem)

        owait()
        owait()

    return gather_kernel


def kernel(x, table):
    n_b1, n_b2 = x.shape
    xp = jnp.pad(x.astype(jnp.int32), ((0, 0), (0, PAD_B2 - n_b2)))
    y = xp.reshape(n_b1 // IDX_LANE, IDX_LANE, PAD_B2 // 8, 8)
    y = y.transpose(2, 0, 3, 1)
    out = _make_gather(n_b1, n_b2)(y, table)
    return jnp.transpose(out, (2, 0, 1))


# speedup vs baseline: 2.1115x; 1.0030x over previous
import functools

import jax
import jax.numpy as jnp
from jax import lax
from jax.experimental import pallas as pl
from jax.experimental.pallas import tpu as pltpu
from jax.experimental.pallas import tpu_sc as plsc

EMBED_DIM = 32
NUM_CORES = 2
NUM_SUBCORES = 16
NW = NUM_CORES * NUM_SUBCORES
IDX_LANE = 128
L = 16
PAD_B2 = 32


@functools.lru_cache(maxsize=None)
def _make_gather(n_b1: int, n_b2: int):
    cols_per_w = n_b1 // NW
    VPB = cols_per_w // IDX_LANE

    mesh = plsc.VectorSubcoreMesh(
        core_axis_name="c", subcore_axis_name="s",
        num_cores=NUM_CORES, num_subcores=NUM_SUBCORES)

    @functools.partial(
        pl.kernel,
        mesh=mesh,
        out_type=jax.ShapeDtypeStruct((n_b2, EMBED_DIM, n_b1), jnp.float32),
        scratch_types=[
            pltpu.VMEM((PAD_B2 // 8, cols_per_w // IDX_LANE, 8, IDX_LANE),
                       jnp.int32),
            pltpu.VMEM((2, cols_per_w, EMBED_DIM), jnp.float32),
            pltpu.VMEM((2, EMBED_DIM, cols_per_w), jnp.float32),
            pltpu.SemaphoreType.DMA,
            pltpu.SemaphoreType.DMA,
        ],
        compiler_params=pltpu.CompilerParams(
            use_tc_tiling_on_sc=False, needs_layout_passes=False),
    )
    def gather_kernel(idx_hbm, table_hbm, out_hbm, idx_v, rows_v, tbuf,
                      gsem, osem):
        wid = lax.axis_index("s") * NUM_CORES + lax.axis_index("c")
        c0 = wid * cols_per_w
        tc0 = wid * (cols_per_w // IDX_LANE)
        pltpu.sync_copy(
            idx_hbm.at[:, pl.ds(tc0, cols_per_w // IDX_LANE)], idx_v)
        iota = lax.iota(jnp.int32, L)

        def fire(b2, slot):
            tr = b2 // 8
            r = b2 % 8
            for k in range(VPB):
                pltpu.async_copy(
                    table_hbm.at[idx_v.at[tr, k, r]],
                    rows_v.at[slot].at[pl.ds(k * IDX_LANE, IDX_LANE)],
                    gsem)

        def drain(slot):
            for k in range(VPB):
                pltpu.make_async_copy(
                    table_hbm.at[idx_v.at[0, 0, 0]],
                    rows_v.at[slot].at[pl.ds(0, IDX_LANE)],
                    gsem).wait()

        def out_dst(b2):
            return out_hbm.at[b2, :, pl.ds(c0, cols_per_w)]

        def owait():
            pltpu.make_async_copy(tbuf.at[0], out_dst(0), osem).wait()

        def transpose(slot):
            @pl.loop(0, cols_per_w // 8)
            def _cg(cg):
                for j in range(8):
                    c = cg * 8 + j
                    cv = jnp.full((L,), 0, jnp.int32) + c
                    for h in range(EMBED_DIM // L):
                        vals = rows_v[slot, c, pl.ds(h * L, L)]
                        plsc.store_scatter(
                            tbuf.at[slot], [iota + h * L, cv], vals)

        fire(0, 0)

        @pl.loop(0, n_b2)
        def _b2(b2):
            slot = lax.rem(b2, 2)
            drain(slot)

            @pl.when(b2 + 1 < n_b2)
            def _fire_next():
                fire(b2 + 1, lax.rem(b2 + 1, 2))

            @pl.when(b2 >= 2)
            def _wait_out():
                owait()

            transpose(slot)
            pltpu.async_copy(tbuf.at[slot], out_dst(b2), osem)

        owait()
        owait()

    return gather_kernel


def kernel(x, table):
    n_b1, n_b2 = x.shape
    xp = jnp.pad(x.astype(jnp.int32), ((0, 0), (0, PAD_B2 - n_b2)))
    y = xp.reshape(n_b1 // IDX_LANE, IDX_LANE, PAD_B2 // 8, 8)
    y = y.transpose(2, 0, 3, 1)
    out = _make_gather(n_b1, n_b2)(y, table)
    return jnp.transpose(out, (2, 0, 1))


# speedup vs baseline: 2.3149x; 1.0963x over previous
import functools

import jax
import jax.numpy as jnp
from jax import lax
from jax.experimental import pallas as pl
from jax.experimental.pallas import tpu as pltpu
from jax.experimental.pallas import tpu_sc as plsc

EMBED_DIM = 32
NUM_CORES = 2
NUM_SUBCORES = 16
NW = NUM_CORES * NUM_SUBCORES
IDX_LANE = 128
L = 16
PAD_B2 = 32


@functools.lru_cache(maxsize=None)
def _make_gather(n_b1: int, n_b2: int):
    cols_per_w = n_b1 // NW
    VPB = cols_per_w // IDX_LANE

    mesh = plsc.VectorSubcoreMesh(
        core_axis_name="c", subcore_axis_name="s",
        num_cores=NUM_CORES, num_subcores=NUM_SUBCORES)

    @functools.partial(
        pl.kernel,
        mesh=mesh,
        out_type=jax.ShapeDtypeStruct((n_b2, n_b1, EMBED_DIM), jnp.float32),
        scratch_types=[
            pltpu.VMEM((PAD_B2 // 8, cols_per_w // IDX_LANE, 8, IDX_LANE),
                       jnp.int32),
            pltpu.VMEM((2, cols_per_w, EMBED_DIM), jnp.float32),
            pltpu.SemaphoreType.DMA,
            pltpu.SemaphoreType.DMA,
        ],
        compiler_params=pltpu.CompilerParams(
            use_tc_tiling_on_sc=False, needs_layout_passes=False),
    )
    def gather_kernel(idx_hbm, table_hbm, out_hbm, idx_v, rows_v,
                      gsem, osem):
        wid = lax.axis_index("s") * NUM_CORES + lax.axis_index("c")
        c0 = wid * cols_per_w
        tc0 = wid * (cols_per_w // IDX_LANE)
        pltpu.sync_copy(
            idx_hbm.at[:, pl.ds(tc0, cols_per_w // IDX_LANE)], idx_v)
        iota = lax.iota(jnp.int32, L)

        def fire(b2, slot):
            tr = b2 // 8
            r = b2 % 8
            for k in range(VPB):
                pltpu.async_copy(
                    table_hbm.at[idx_v.at[tr, k, r]],
                    rows_v.at[slot].at[pl.ds(k * IDX_LANE, IDX_LANE)],
                    gsem)

        def drain(slot):
            for k in range(VPB):
                pltpu.make_async_copy(
                    table_hbm.at[idx_v.at[0, 0, 0]],
                    rows_v.at[slot].at[pl.ds(0, IDX_LANE)],
                    gsem).wait()

        def out_dst(b2):
            return out_hbm.at[b2, pl.ds(c0, cols_per_w)]

        def owait():
            pltpu.make_async_copy(rows_v.at[0], out_dst(0), osem).wait()

        fire(0, 0)

        @pl.loop(0, n_b2)
        def _b2(b2):
            slot = lax.rem(b2, 2)
            drain(slot)

            @pl.when(b2 + 1 < n_b2)
            def _fire_next():
                fire(b2 + 1, lax.rem(b2 + 1, 2))

            @pl.when(b2 >= 2)
            def _wait_out():
                owait()

            pltpu.async_copy(rows_v.at[slot], out_dst(b2), osem)

        owait()
        owait()

    return gather_kernel


def kernel(x, table):
    n_b1, n_b2 = x.shape
    xp = jnp.pad(x.astype(jnp.int32), ((0, 0), (0, PAD_B2 - n_b2)))
    y = xp.reshape(n_b1 // IDX_LANE, IDX_LANE, PAD_B2 // 8, 8)
    y = y.transpose(2, 0, 3, 1)
    out = _make_gather(n_b1, n_b2)(y, table)
    return jnp.transpose(out, (1, 0, 2))
